# Initial kernel scaffold; baseline (speedup 1.0000x reference)
import jax
import jax.numpy as jnp
from jax.experimental import pallas as pl


def kernel(x, adj, W1, b1, W2, b2, W3, b3):
    raise NotImplementedError("write your pallas kernel here")


# speedup vs baseline: 2.0682x; 2.0682x over previous
import functools

import jax
import jax.numpy as jnp
from jax.experimental import pallas as pl

N = 4096
BM = 512


def _layer1_kernel(adj_ref, x_ref, w1_ref, b1_ref, w2_ref, out_ref):
    ax = jnp.dot(adj_ref[...], x_ref[...], preferred_element_type=jnp.float32)
    h = jnp.dot(ax, w1_ref[...], preferred_element_type=jnp.float32) + b1_ref[...]
    h = jnp.where(h > 0, h, 0.01 * h)
    out_ref[...] = jnp.dot(h, w2_ref[...], preferred_element_type=jnp.float32)


def _layer2_kernel(adj_ref, s_ref, b2_ref, w3_ref, out_ref):
    h = jnp.dot(adj_ref[...], s_ref[...], preferred_element_type=jnp.float32)
    h = jnp.maximum(h + b2_ref[...], 0.0)
    out_ref[...] = jnp.dot(h, w3_ref[...], preferred_element_type=jnp.float32)


def _layer3_kernel(adj_ref, s_ref, b3_ref, out_ref):
    h = jnp.dot(adj_ref[...], s_ref[...], preferred_element_type=jnp.float32)
    out_ref[...] = jnp.maximum(h + b3_ref[...], 0.0)


def _row_grid(n_cols_in, n_cols_out, n_extra_full):
    grid = (N // BM,)
    adj_spec = pl.BlockSpec((BM, N), lambda i: (i, 0))
    full = lambda shape: pl.BlockSpec(shape, lambda i: (0,) * len(shape))
    out_spec = pl.BlockSpec((BM, n_cols_out), lambda i: (i, 0))
    return grid, adj_spec, full, out_spec


@jax.jit
def kernel(x, adj, W1, b1, W2, b2, W3, b3):
    in_feat = x.shape[1]
    nhid1 = W1.shape[1]
    nhid2 = W2.shape[1]
    out_feat = W3.shape[1]
    b1r = b1.reshape(1, nhid1)
    b2r = b2.reshape(1, nhid2)
    b3r = b3.reshape(1, out_feat)

    grid, adj_spec, full, _ = _row_grid(in_feat, nhid2, 0)

    s2 = pl.pallas_call(
        _layer1_kernel,
        grid=grid,
        in_specs=[
            adj_spec,
            full((N, in_feat)),
            full((in_feat, nhid1)),
            full((1, nhid1)),
            full((nhid1, nhid2)),
        ],
        out_specs=pl.BlockSpec((BM, nhid2), lambda i: (i, 0)),
        out_shape=jax.ShapeDtypeStruct((N, nhid2), jnp.float32),
    )(adj, x, W1, b1r, W2)

    s3 = pl.pallas_call(
        _layer2_kernel,
        grid=grid,
        in_specs=[
            adj_spec,
            full((N, nhid2)),
            full((1, nhid2)),
            full((nhid2, out_feat)),
        ],
        out_specs=pl.BlockSpec((BM, out_feat), lambda i: (i, 0)),
        out_shape=jax.ShapeDtypeStruct((N, out_feat), jnp.float32),
    )(adj, s2, b2r, W3)

    out = pl.pallas_call(
        _layer3_kernel,
        grid=grid,
        in_specs=[
            adj_spec,
            full((N, out_feat)),
            full((1, out_feat)),
        ],
        out_specs=pl.BlockSpec((BM, out_feat), lambda i: (i, 0)),
        out_shape=jax.ShapeDtypeStruct((N, out_feat), jnp.float32),
    )(adj, s3, b3r)

    return out


# speedup vs baseline: 2.2051x; 1.0662x over previous
import functools

import jax
import jax.numpy as jnp
from jax.experimental import pallas as pl

N = 4096
BM = 512


def _bf(a):
    return a.astype(jnp.bfloat16)


def _layer1_kernel(adj_ref, x_ref, w1_ref, b1_ref, w2_ref, s2_ref, adjb_ref):
    adjb = _bf(adj_ref[...])
    adjb_ref[...] = adjb
    ax = jnp.dot(adjb, x_ref[...], preferred_element_type=jnp.float32)
    h = jnp.dot(_bf(ax), w1_ref[...], preferred_element_type=jnp.float32)
    h = h + b1_ref[...]
    h = jnp.where(h > 0, h, 0.01 * h)
    s2_ref[...] = _bf(jnp.dot(_bf(h), w2_ref[...], preferred_element_type=jnp.float32))


def _layer2_kernel(adjb_ref, s_ref, b2_ref, w3_ref, out_ref):
    h = jnp.dot(adjb_ref[...], s_ref[...], preferred_element_type=jnp.float32)
    h = jnp.maximum(h + b2_ref[...], 0.0)
    out_ref[...] = _bf(jnp.dot(_bf(h), w3_ref[...], preferred_element_type=jnp.float32))


def _layer3_kernel(adjb_ref, s_ref, b3_ref, out_ref):
    h = jnp.dot(adjb_ref[...], s_ref[...], preferred_element_type=jnp.float32)
    out_ref[...] = jnp.maximum(h + b3_ref[...], 0.0)


@jax.jit
def kernel(x, adj, W1, b1, W2, b2, W3, b3):
    in_feat = x.shape[1]
    nhid1 = W1.shape[1]
    nhid2 = W2.shape[1]
    out_feat = W3.shape[1]
    b1r = b1.reshape(1, nhid1)
    b2r = b2.reshape(1, nhid2)
    b3r = b3.reshape(1, out_feat)

    grid = (N // BM,)
    adj_spec = pl.BlockSpec((BM, N), lambda i: (i, 0))
    full = lambda shape: pl.BlockSpec(shape, lambda i: (0,) * len(shape))

    s2, adj_bf = pl.pallas_call(
        _layer1_kernel,
        grid=grid,
        in_specs=[
            adj_spec,
            full((N, in_feat)),
            full((in_feat, nhid1)),
            full((1, nhid1)),
            full((nhid1, nhid2)),
        ],
        out_specs=(
            pl.BlockSpec((BM, nhid2), lambda i: (i, 0)),
            pl.BlockSpec((BM, N), lambda i: (i, 0)),
        ),
        out_shape=(
            jax.ShapeDtypeStruct((N, nhid2), jnp.bfloat16),
            jax.ShapeDtypeStruct((N, N), jnp.bfloat16),
        ),
    )(adj, _bf(x), _bf(W1), b1r, _bf(W2))

    s3 = pl.pallas_call(
        _layer2_kernel,
        grid=grid,
        in_specs=[
            adj_spec,
            full((N, nhid2)),
            full((1, nhid2)),
            full((nhid2, out_feat)),
        ],
        out_specs=pl.BlockSpec((BM, out_feat), lambda i: (i, 0)),
        out_shape=jax.ShapeDtypeStruct((N, out_feat), jnp.bfloat16),
    )(adj_bf, s2, b2r, _bf(W3))

    out = pl.pallas_call(
        _layer3_kernel,
        grid=grid,
        in_specs=[
            adj_spec,
            full((N, out_feat)),
            full((1, out_feat)),
        ],
        out_specs=pl.BlockSpec((BM, out_feat), lambda i: (i, 0)),
        out_shape=jax.ShapeDtypeStruct((N, out_feat), jnp.float32),
    )(adj_bf, s3, b3r)

    return out


# speedup vs baseline: 2.2963x; 1.0414x over previous
import functools

import jax
import jax.numpy as jnp
from jax.experimental import pallas as pl
from jax.experimental.pallas import tpu as pltpu

N = 4096
BM = 256
G = N // BM


def _bf(a):
    return a.astype(jnp.bfloat16)


def _mega_kernel(adj_ref, x_ref, w1_ref, b1_ref, w2_ref, b2_ref, w3_ref,
                 b3_ref, out_ref, adjb_ref, s2_ref, s3_ref):
    i = pl.program_id(0)

    @pl.when(i < G)
    def _phase0():
        adjb = _bf(adj_ref[...])
        adjb_ref[pl.ds(i * BM, BM), :] = adjb
        ax = jnp.dot(adjb, x_ref[...], preferred_element_type=jnp.float32)
        h = jnp.dot(_bf(ax), w1_ref[...], preferred_element_type=jnp.float32)
        h = h + b1_ref[...]
        h = jnp.where(h > 0, h, 0.01 * h)
        s2 = jnp.dot(_bf(h), w2_ref[...], preferred_element_type=jnp.float32)
        s2_ref[pl.ds(i * BM, BM), :] = _bf(s2)

    @pl.when(jnp.logical_and(i >= G, i < 2 * G))
    def _phase1():
        j = i - G
        a = adjb_ref[pl.ds(j * BM, BM), :]
        h = jnp.dot(a, s2_ref[...], preferred_element_type=jnp.float32)
        h = jnp.maximum(h + b2_ref[...], 0.0)
        s3 = jnp.dot(_bf(h), w3_ref[...], preferred_element_type=jnp.float32)
        s3_ref[pl.ds(j * BM, BM), :] = _bf(s3)

    @pl.when(i >= 2 * G)
    def _phase2():
        j = i - 2 * G
        a = adjb_ref[pl.ds(j * BM, BM), :]
        h = jnp.dot(a, s3_ref[...], preferred_element_type=jnp.float32)
        out_ref[...] = jnp.maximum(h + b3_ref[...], 0.0)


@jax.jit
def kernel(x, adj, W1, b1, W2, b2, W3, b3):
    in_feat = x.shape[1]
    nhid1 = W1.shape[1]
    nhid2 = W2.shape[1]
    out_feat = W3.shape[1]
    b1r = b1.reshape(1, nhid1)
    b2r = b2.reshape(1, nhid2)
    b3r = b3.reshape(1, out_feat)

    full = lambda shape: pl.BlockSpec(shape, lambda i: (0,) * len(shape))

    out = pl.pallas_call(
        _mega_kernel,
        grid=(3 * G,),
        in_specs=[
            pl.BlockSpec((BM, N), lambda i: (jnp.where(i < G, i, 0), 0)),
            full((N, in_feat)),
            full((in_feat, nhid1)),
            full((1, nhid1)),
            full((nhid1, nhid2)),
            full((1, nhid2)),
            full((nhid2, out_feat)),
            full((1, out_feat)),
        ],
        out_specs=pl.BlockSpec(
            (BM, out_feat), lambda i: (jnp.where(i >= 2 * G, i - 2 * G, 0), 0)
        ),
        out_shape=jax.ShapeDtypeStruct((N, out_feat), jnp.float32),
        scratch_shapes=[
            pltpu.VMEM((N, N), jnp.bfloat16),
            pltpu.VMEM((N, nhid2), jnp.bfloat16),
            pltpu.VMEM((N, out_feat), jnp.bfloat16),
        ],
        compiler_params=pltpu.CompilerParams(
            vmem_limit_bytes=112 * 1024 * 1024,
        ),
    )(adj, _bf(x), _bf(W1), b1r, _bf(W2), b2r, _bf(W3), b3r)

    return out


# speedup vs baseline: 2.5123x; 1.0941x over previous
import functools

import jax
import jax.numpy as jnp
from jax.experimental import pallas as pl
from jax.experimental.pallas import tpu as pltpu

N = 4096
BM = 512
G = N // BM


def _bf(a):
    return a.astype(jnp.bfloat16)


def _mega_kernel(adj_ref, x_ref, w1_ref, b1_ref, w2_ref, b2_ref, w3_ref,
                 b3_ref, out_ref, adjb_ref, s2_ref, s3_ref):
    i = pl.program_id(0)

    @pl.when(i < G)
    def _phase0():
        adjb = _bf(adj_ref[...])
        adjb_ref[pl.ds(i * BM, BM), :] = adjb
        ax = jnp.dot(adjb, x_ref[...], preferred_element_type=jnp.float32)
        h = jnp.dot(_bf(ax), w1_ref[...], preferred_element_type=jnp.float32)
        h = h + b1_ref[...]
        h = jnp.where(h > 0, h, 0.01 * h)
        s2 = jnp.dot(_bf(h), w2_ref[...], preferred_element_type=jnp.float32)
        s2_ref[pl.ds(i * BM, BM), :] = _bf(s2)

    @pl.when(jnp.logical_and(i >= G, i < 2 * G))
    def _phase1():
        j = i - G
        a = adjb_ref[pl.ds(j * BM, BM), :]
        h = jnp.dot(a, s2_ref[...], preferred_element_type=jnp.float32)
        h = jnp.maximum(h + b2_ref[...], 0.0)
        s3 = jnp.dot(_bf(h), w3_ref[...], preferred_element_type=jnp.float32)
        s3_ref[pl.ds(j * BM, BM), :] = _bf(s3)

    @pl.when(i >= 2 * G)
    def _phase2():
        j = i - 2 * G
        a = adjb_ref[pl.ds(j * BM, BM), :]
        h = jnp.dot(a, s3_ref[...], preferred_element_type=jnp.float32)
        out_ref[...] = jnp.maximum(h + b3_ref[...], 0.0)


@jax.jit
def kernel(x, adj, W1, b1, W2, b2, W3, b3):
    in_feat = x.shape[1]
    nhid1 = W1.shape[1]
    nhid2 = W2.shape[1]
    out_feat = W3.shape[1]
    b1r = b1.reshape(1, nhid1)
    b2r = b2.reshape(1, nhid2)
    b3r = b3.reshape(1, out_feat)

    full = lambda shape: pl.BlockSpec(shape, lambda i: (0,) * len(shape))

    out = pl.pallas_call(
        _mega_kernel,
        grid=(3 * G,),
        in_specs=[
            pl.BlockSpec((BM, N), lambda i: (jnp.where(i < G, i, 0), 0)),
            full((N, in_feat)),
            full((in_feat, nhid1)),
            full((1, nhid1)),
            full((nhid1, nhid2)),
            full((1, nhid2)),
            full((nhid2, out_feat)),
            full((1, out_feat)),
        ],
        out_specs=pl.BlockSpec(
            (BM, out_feat), lambda i: (jnp.where(i >= 2 * G, i - 2 * G, 0), 0)
        ),
        out_shape=jax.ShapeDtypeStruct((N, out_feat), jnp.float32),
        scratch_shapes=[
            pltpu.VMEM((N, N), jnp.bfloat16),
            pltpu.VMEM((N, nhid2), jnp.bfloat16),
            pltpu.VMEM((N, out_feat), jnp.bfloat16),
        ],
        compiler_params=pltpu.CompilerParams(
            vmem_limit_bytes=112 * 1024 * 1024,
        ),
    )(adj, _bf(x), _bf(W1), b1r, _bf(W2), b2r, _bf(W3), b3r)

    return out


# speedup vs baseline: 2.5316x; 1.0077x over previous
import jax
import jax.numpy as jnp
from jax.experimental import pallas as pl
from jax.experimental.pallas import tpu as pltpu

N = 4096
BM = 512
G = N // BM


def _bf(a):
    return a.astype(jnp.bfloat16)


def _mega_kernel(adj_ref, x_ref, w1_ref, b1_ref, w2_ref, b2_ref, w3_ref,
                 b3_ref, out_ref, adjb_ref, s2_ref, s3_ref):
    i = pl.program_id(0)

    @pl.when(i < G)
    def _phase0():
        adjb = _bf(adj_ref[...])
        adjb_ref[pl.ds(i * BM, BM), :] = adjb
        ax = jnp.dot(adjb, x_ref[...], preferred_element_type=jnp.float32)
        h = jnp.dot(_bf(ax), w1_ref[...], preferred_element_type=jnp.float32)
        h = h + b1_ref[...]
        h = jnp.where(h > 0, h, 0.01 * h)
        s2 = jnp.dot(_bf(h), w2_ref[...], preferred_element_type=jnp.float32)
        s2_ref[pl.ds(i * BM, BM), :] = _bf(s2)

    @pl.when(jnp.logical_and(i >= G, i < 2 * G))
    def _phase1():
        j = i - G
        a = adjb_ref[pl.ds(j * BM, BM), :]
        h = jnp.dot(a, s2_ref[...], preferred_element_type=jnp.float32)
        h = jnp.maximum(h + b2_ref[...], 0.0)
        s3 = jnp.dot(_bf(h), w3_ref[...], preferred_element_type=jnp.float32)
        s3_ref[pl.ds(j * BM, BM), :] = _bf(s3)

    @pl.when(i >= 2 * G)
    def _phase2():
        j = i - 2 * G
        a = adjb_ref[pl.ds(j * BM, BM), :]
        h = jnp.dot(a, s3_ref[...], preferred_element_type=jnp.float32)
        out_ref[...] = jnp.maximum(h + b3_ref[...], 0.0)


@jax.jit
def kernel(x, adj, W1, b1, W2, b2, W3, b3):
    in_feat = x.shape[1]
    nhid1 = W1.shape[1]
    nhid2 = W2.shape[1]
    out_feat = W3.shape[1]
    b1r = b1.reshape(1, nhid1)
    b2r = b2.reshape(1, nhid2)
    b3r = b3.reshape(1, out_feat)

    full = lambda shape: pl.BlockSpec(shape, lambda i: (0,) * len(shape))

    out = pl.pallas_call(
        _mega_kernel,
        grid=(3 * G,),
        in_specs=[
            pl.BlockSpec((BM, N), lambda i: (jnp.where(i < G, i, 0), 0)),
            full((N, in_feat)),
            full((in_feat, nhid1)),
            full((1, nhid1)),
            full((nhid1, nhid2)),
            full((1, nhid2)),
            full((nhid2, out_feat)),
            full((1, out_feat)),
        ],
        out_specs=pl.BlockSpec(
            (BM, out_feat), lambda i: (jnp.where(i >= 2 * G, i - 2 * G, 0), 0)
        ),
        out_shape=jax.ShapeDtypeStruct((N, out_feat), jnp.float32),
        scratch_shapes=[
            pltpu.VMEM((N, N), jnp.bfloat16),
            pltpu.VMEM((N, nhid2), jnp.bfloat16),
            pltpu.VMEM((N, out_feat), jnp.bfloat16),
        ],
        compiler_params=pltpu.CompilerParams(
            vmem_limit_bytes=112 * 1024 * 1024,
        ),
    )(adj, _bf(x), _bf(W1), b1r, _bf(W2), b2r, _bf(W3), b3r)

    return out
